# baseline (speedup 1.0000x reference)
import jax
import jax.numpy as jnp
from jax import lax
from jax.experimental import pallas as pl
from jax.experimental.pallas import tpu as pltpu
from jax.experimental.pallas import tpu_sc as plsc

BATCH = 16384
DIM = 64
NW = 32
ROWS = BATCH // NW
W = 64
NWIN = ROWS // W
LANES = 16


def _transe_kernel(h_hbm, t_hbm, r_hbm, ent_hbm, rel_hbm, o_hbm,
                   ivh, ivt, ivr,
                   ph0, ph1, pt0, pt1, pr0, pr1,
                   hbuf0, hbuf1, tbuf0, tbuf1, rbuf0, rbuf1,
                   obuf0, obuf1,
                   sem0, sem1, osem0, osem1):
    wid = lax.axis_index("core") * 16 + lax.axis_index("subcore")
    base = wid * ROWS

    pltpu.sync_copy(h_hbm.at[pl.ds(base, ROWS)], ivh)
    pltpu.sync_copy(t_hbm.at[pl.ds(base, ROWS)], ivt)
    pltpu.sync_copy(r_hbm.at[pl.ds(base, ROWS)], ivr)

    def issue(g, ph, pt, pr, hbuf, tbuf, rbuf, sem):
        rb = g * W
        for k in range(W // LANES):
            s = pl.ds(rb + k * LANES, LANES)
            d = pl.ds(k * LANES, LANES)
            ph[d] = lax.shift_right_logical(ivh[s], 1)
            pt[d] = lax.shift_right_logical(ivt[s], 1)
            pr[d] = lax.shift_right_logical(ivr[s], 1)
        pltpu.async_copy(ent_hbm.at[ph], hbuf, sem)
        pltpu.async_copy(ent_hbm.at[pt], tbuf, sem)
        pltpu.async_copy(rel_hbm.at[pr], rbuf, sem)

    def drain(ph, pt, pr, hbuf, tbuf, rbuf, sem):
        pltpu.make_async_copy(ent_hbm.at[ph], hbuf, sem).wait()
        pltpu.make_async_copy(ent_hbm.at[pt], tbuf, sem).wait()
        pltpu.make_async_copy(rel_hbm.at[pr], rbuf, sem).wait()

    def compute(g, hbuf, tbuf, rbuf, obuf, osem):
        rb = g * W
        pltpu.make_async_copy(obuf, o_hbm.at[pl.ds(base, W)], osem).wait()

        @pl.loop(0, W // LANES)
        def _(k):
            kb = k * LANES
            hv = ivh[pl.ds(rb + kb, LANES)] & 1
            tv = ivt[pl.ds(rb + kb, LANES)] & 1
            rv = ivr[pl.ds(rb + kb, LANES)] & 1
            for w in range(LANES):
                hc = hv[w] * DIM
                tc = tv[w] * DIM
                rc = rv[w] * DIM
                for j in range(DIM // LANES):
                    jo = j * LANES
                    obuf.at[kb + w, pl.ds(jo, LANES)][...] = (
                        hbuf.at[kb + w, pl.ds(hc + jo, LANES)][...]
                        + rbuf.at[kb + w, pl.ds(rc + jo, LANES)][...]
                        - tbuf.at[kb + w, pl.ds(tc + jo, LANES)][...]
                    )

        pltpu.async_copy(obuf, o_hbm.at[pl.ds(base + rb, W)], osem)

    pltpu.async_copy(obuf0, o_hbm.at[pl.ds(base, W)], osem0)
    pltpu.async_copy(obuf1, o_hbm.at[pl.ds(base, W)], osem1)
    issue(0, ph0, pt0, pr0, hbuf0, tbuf0, rbuf0, sem0)

    @pl.loop(0, NWIN, step=2)
    def _(g):
        issue(g + 1, ph1, pt1, pr1, hbuf1, tbuf1, rbuf1, sem1)
        drain(ph0, pt0, pr0, hbuf0, tbuf0, rbuf0, sem0)
        compute(g, hbuf0, tbuf0, rbuf0, obuf0, osem0)

        @pl.when(g + 2 < NWIN)
        def _():
            issue(g + 2, ph0, pt0, pr0, hbuf0, tbuf0, rbuf0, sem0)

        drain(ph1, pt1, pr1, hbuf1, tbuf1, rbuf1, sem1)
        compute(g + 1, hbuf1, tbuf1, rbuf1, obuf1, osem1)

    pltpu.make_async_copy(obuf0, o_hbm.at[pl.ds(base, W)], osem0).wait()
    pltpu.make_async_copy(obuf1, o_hbm.at[pl.ds(base, W)], osem1).wait()


@jax.jit
def kernel(h_list, t_list, r_list, ent_embeddings, rel_embeddings):
    n_ent, dim = ent_embeddings.shape
    n_rel = rel_embeddings.shape[0]
    mesh = plsc.VectorSubcoreMesh(core_axis_name="core",
                                  subcore_axis_name="subcore")
    idxbuf = pltpu.VMEM((ROWS,), jnp.int32)
    pairbuf = pltpu.VMEM((W,), jnp.int32)
    rowbuf = pltpu.VMEM((W, 2 * DIM), ent_embeddings.dtype)
    outbuf = pltpu.VMEM((W, DIM), ent_embeddings.dtype)
    run = pl.kernel(
        _transe_kernel,
        out_type=jax.ShapeDtypeStruct((BATCH, DIM), ent_embeddings.dtype),
        mesh=mesh,
        scratch_types=[
            idxbuf, idxbuf, idxbuf,
            pairbuf, pairbuf, pairbuf, pairbuf, pairbuf, pairbuf,
            rowbuf, rowbuf, rowbuf, rowbuf, rowbuf, rowbuf,
            outbuf, outbuf,
            pltpu.SemaphoreType.DMA,
            pltpu.SemaphoreType.DMA,
            pltpu.SemaphoreType.DMA,
            pltpu.SemaphoreType.DMA,
        ],
    )
    return run(
        h_list.astype(jnp.int32),
        t_list.astype(jnp.int32),
        r_list.astype(jnp.int32),
        ent_embeddings.reshape(n_ent // 2, 2 * dim),
        rel_embeddings.reshape(n_rel // 2, 2 * dim),
    )


# speedup vs baseline: 1.9633x; 1.9633x over previous
import jax
import jax.numpy as jnp
from jax import lax
from jax.experimental import pallas as pl
from jax.experimental.pallas import tpu as pltpu
from jax.experimental.pallas import tpu_sc as plsc

BATCH = 16384
DIM = 64
NW = 32
ROWS = BATCH // NW
C = 16
NCHUNK = ROWS // C
LANES = 16


def _transe_kernel(h_hbm, t_hbm, r_hbm, ent_hbm, rel_hbm, o_hbm,
                   ivh, ivt, ivr,
                   hbuf0, hbuf1, tbuf0, tbuf1, rbuf0, rbuf1,
                   obuf0, obuf1,
                   sem0, sem1, osem0, osem1):
    wid = lax.axis_index("core") * 16 + lax.axis_index("subcore")
    base = wid * ROWS

    pltpu.sync_copy(h_hbm.at[pl.ds(base, ROWS)], ivh)
    pltpu.sync_copy(t_hbm.at[pl.ds(base, ROWS)], ivt)
    pltpu.sync_copy(r_hbm.at[pl.ds(base, ROWS)], ivr)

    def issue(g, hbuf, tbuf, rbuf, sem):
        th = lax.shift_right_logical(ivh[pl.ds(g * C, C)], 3)
        tt = lax.shift_right_logical(ivt[pl.ds(g * C, C)], 3)
        tr = lax.shift_right_logical(ivr[pl.ds(g * C, C)], 3)
        for w in range(C):
            pltpu.async_copy(ent_hbm.at[pl.ds(th[w], 1)],
                             hbuf.at[pl.ds(w, 1)], sem)
            pltpu.async_copy(ent_hbm.at[pl.ds(tt[w], 1)],
                             tbuf.at[pl.ds(w, 1)], sem)
            pltpu.async_copy(rel_hbm.at[pl.ds(tr[w], 1)],
                             rbuf.at[pl.ds(w, 1)], sem)

    def drain(hbuf, tbuf, rbuf, sem):
        pltpu.make_async_copy(ent_hbm.at[pl.ds(0, C)], hbuf, sem).wait()
        pltpu.make_async_copy(ent_hbm.at[pl.ds(0, C)], tbuf, sem).wait()
        pltpu.make_async_copy(rel_hbm.at[pl.ds(0, C)], rbuf, sem).wait()

    def compute(g, hbuf, tbuf, rbuf, obuf, osem):
        pltpu.make_async_copy(obuf, o_hbm.at[pl.ds(base, C)], osem).wait()
        hv = ivh[pl.ds(g * C, C)] & 7
        tv = ivt[pl.ds(g * C, C)] & 7
        rv = ivr[pl.ds(g * C, C)] & 7
        for w in range(C):
            hs = hv[w]
            ts = tv[w]
            rs = rv[w]
            for j in range(DIM // LANES):
                s = pl.ds(j * LANES, LANES)
                obuf.at[w, s][...] = (
                    hbuf.at[w, hs, s][...]
                    + rbuf.at[w, rs, s][...]
                    - tbuf.at[w, ts, s][...]
                )
        pltpu.async_copy(obuf, o_hbm.at[pl.ds(base + g * C, C)], osem)

    pltpu.async_copy(obuf0, o_hbm.at[pl.ds(base, C)], osem0)
    pltpu.async_copy(obuf1, o_hbm.at[pl.ds(base + C, C)], osem1)
    issue(0, hbuf0, tbuf0, rbuf0, sem0)

    @pl.loop(0, NCHUNK, step=2)
    def _(g):
        issue(g + 1, hbuf1, tbuf1, rbuf1, sem1)
        drain(hbuf0, tbuf0, rbuf0, sem0)
        compute(g, hbuf0, tbuf0, rbuf0, obuf0, osem0)

        @pl.when(g + 2 < NCHUNK)
        def _():
            issue(g + 2, hbuf0, tbuf0, rbuf0, sem0)

        drain(hbuf1, tbuf1, rbuf1, sem1)
        compute(g + 1, hbuf1, tbuf1, rbuf1, obuf1, osem1)

    pltpu.make_async_copy(obuf0, o_hbm.at[pl.ds(base, C)], osem0).wait()
    pltpu.make_async_copy(obuf1, o_hbm.at[pl.ds(base, C)], osem1).wait()


@jax.jit
def kernel(h_list, t_list, r_list, ent_embeddings, rel_embeddings):
    n_ent, dim = ent_embeddings.shape
    n_rel = rel_embeddings.shape[0]
    mesh = plsc.VectorSubcoreMesh(core_axis_name="core",
                                  subcore_axis_name="subcore")
    idxbuf = pltpu.VMEM((ROWS,), jnp.int32)
    fbuf = pltpu.VMEM((C, 8, DIM), ent_embeddings.dtype)
    outbuf = pltpu.VMEM((C, DIM), ent_embeddings.dtype)
    run = pl.kernel(
        _transe_kernel,
        out_type=jax.ShapeDtypeStruct((BATCH, DIM), ent_embeddings.dtype),
        mesh=mesh,
        scratch_types=[
            idxbuf, idxbuf, idxbuf,
            fbuf, fbuf, fbuf, fbuf, fbuf, fbuf,
            outbuf, outbuf,
            pltpu.SemaphoreType.DMA,
            pltpu.SemaphoreType.DMA,
            pltpu.SemaphoreType.DMA,
            pltpu.SemaphoreType.DMA,
        ],
    )
    return run(
        h_list.astype(jnp.int32),
        t_list.astype(jnp.int32),
        r_list.astype(jnp.int32),
        ent_embeddings.reshape(n_ent // 8, 8, dim),
        rel_embeddings.reshape(n_rel // 8, 8, dim),
    )


# speedup vs baseline: 2.4094x; 1.2272x over previous
import jax
import jax.numpy as jnp
from jax import lax
from jax.experimental import pallas as pl
from jax.experimental.pallas import tpu as pltpu
from jax.experimental.pallas import tpu_sc as plsc

BATCH = 16384
DIM = 64
NW = 32
ROWS = BATCH // NW
C = 16
NCHUNK = ROWS // C
LANES = 16


def _transe_kernel(h_hbm, t_hbm, r_hbm, ent_hbm, rel_hbm, o_hbm,
                   ivh, ivt, ivr,
                   pr0, pr1,
                   hbuf0, hbuf1, tbuf0, tbuf1, rbuf0, rbuf1,
                   obuf0, obuf1,
                   sem0, sem1, osem0, osem1):
    wid = lax.axis_index("core") * 16 + lax.axis_index("subcore")
    base = wid * ROWS

    pltpu.sync_copy(h_hbm.at[pl.ds(base, ROWS)], ivh)
    pltpu.sync_copy(t_hbm.at[pl.ds(base, ROWS)], ivt)
    pltpu.sync_copy(r_hbm.at[pl.ds(base, ROWS)], ivr)

    def issue(g, pr, hbuf, tbuf, rbuf, sem):
        hv = ivh[pl.ds(g * C, C)]
        tv = ivt[pl.ds(g * C, C)]
        pr[...] = lax.shift_right_logical(ivr[pl.ds(g * C, C)], 1)
        pltpu.async_copy(rel_hbm.at[pr], rbuf, sem)
        for w in range(C):
            ih = hv[w]
            it = tv[w]
            pltpu.async_copy(
                ent_hbm.at[pl.ds(lax.shift_right_logical(ih, 3), 1),
                           pl.ds(ih & 7, 1)],
                hbuf.at[pl.ds(w, 1)], sem)
            pltpu.async_copy(
                ent_hbm.at[pl.ds(lax.shift_right_logical(it, 3), 1),
                           pl.ds(it & 7, 1)],
                tbuf.at[pl.ds(w, 1)], sem)

    def drain(pr, hbuf, tbuf, rbuf, sem):
        pltpu.make_async_copy(rel_hbm.at[pr], rbuf, sem).wait()
        pltpu.make_async_copy(ent_hbm.at[pl.ds(0, C), pl.ds(0, 1)],
                              hbuf, sem).wait()
        pltpu.make_async_copy(ent_hbm.at[pl.ds(0, C), pl.ds(0, 1)],
                              tbuf, sem).wait()

    def compute(g, hbuf, tbuf, rbuf, obuf, osem):
        pltpu.make_async_copy(obuf, o_hbm.at[pl.ds(base, C)], osem).wait()
        rv = ivr[pl.ds(g * C, C)] & 1
        for w in range(C):
            rc = rv[w] * DIM
            for j in range(DIM // LANES):
                jo = j * LANES
                s = pl.ds(jo, LANES)
                obuf.at[w, s][...] = (
                    hbuf.at[w, 0, s][...]
                    + rbuf.at[w, pl.ds(rc + jo, LANES)][...]
                    - tbuf.at[w, 0, s][...]
                )
        pltpu.async_copy(obuf, o_hbm.at[pl.ds(base + g * C, C)], osem)

    pltpu.async_copy(obuf0, o_hbm.at[pl.ds(base, C)], osem0)
    pltpu.async_copy(obuf1, o_hbm.at[pl.ds(base + C, C)], osem1)
    issue(0, pr0, hbuf0, tbuf0, rbuf0, sem0)

    @pl.loop(0, NCHUNK, step=2)
    def _(g):
        issue(g + 1, pr1, hbuf1, tbuf1, rbuf1, sem1)
        drain(pr0, hbuf0, tbuf0, rbuf0, sem0)
        compute(g, hbuf0, tbuf0, rbuf0, obuf0, osem0)

        @pl.when(g + 2 < NCHUNK)
        def _():
            issue(g + 2, pr0, hbuf0, tbuf0, rbuf0, sem0)

        drain(pr1, hbuf1, tbuf1, rbuf1, sem1)
        compute(g + 1, hbuf1, tbuf1, rbuf1, obuf1, osem1)

    pltpu.make_async_copy(obuf0, o_hbm.at[pl.ds(base, C)], osem0).wait()
    pltpu.make_async_copy(obuf1, o_hbm.at[pl.ds(base, C)], osem1).wait()


@jax.jit
def kernel(h_list, t_list, r_list, ent_embeddings, rel_embeddings):
    n_ent, dim = ent_embeddings.shape
    n_rel = rel_embeddings.shape[0]
    mesh = plsc.VectorSubcoreMesh(core_axis_name="core",
                                  subcore_axis_name="subcore")
    idxbuf = pltpu.VMEM((ROWS,), jnp.int32)
    pairbuf = pltpu.VMEM((C,), jnp.int32)
    rowbuf = pltpu.VMEM((C, 1, DIM), ent_embeddings.dtype)
    relbuf = pltpu.VMEM((C, 2 * DIM), ent_embeddings.dtype)
    outbuf = pltpu.VMEM((C, DIM), ent_embeddings.dtype)
    run = pl.kernel(
        _transe_kernel,
        out_type=jax.ShapeDtypeStruct((BATCH, DIM), ent_embeddings.dtype),
        mesh=mesh,
        scratch_types=[
            idxbuf, idxbuf, idxbuf,
            pairbuf, pairbuf,
            rowbuf, rowbuf, rowbuf, rowbuf, relbuf, relbuf,
            outbuf, outbuf,
            pltpu.SemaphoreType.DMA,
            pltpu.SemaphoreType.DMA,
            pltpu.SemaphoreType.DMA,
            pltpu.SemaphoreType.DMA,
        ],
    )
    return run(
        h_list.astype(jnp.int32),
        t_list.astype(jnp.int32),
        r_list.astype(jnp.int32),
        ent_embeddings.reshape(n_ent // 8, 8, dim),
        rel_embeddings.reshape(n_rel // 2, 2 * dim),
    )
